# baseline (speedup 1.0000x reference)
import jax
import jax.numpy as jnp
from jax.experimental import pallas as pl

_DEPTH = 1000
_ROWS = 4096
_COLS = 26
_BR = 16


def _body(ids_ref, out_ref):
    ids = ids_ref[...]
    cols = jax.lax.broadcasted_iota(jnp.int32, (_BR, _COLS, _DEPTH), 2)
    out_ref[...] = (ids[:, :, None] == cols).astype(jnp.float32)


def kernel(inputs):
    ids = inputs.astype(jnp.int32)
    return pl.pallas_call(
        _body,
        grid=(_ROWS // _BR,),
        in_specs=[pl.BlockSpec((_BR, _COLS), lambda i: (i, 0))],
        out_specs=pl.BlockSpec((_BR, _COLS, _DEPTH), lambda i: (i, 0, 0)),
        out_shape=jax.ShapeDtypeStruct((_ROWS, _COLS, _DEPTH), jnp.float32),
    )(ids)
